# baseline (speedup 1.0000x reference)
import functools

import jax
import jax.numpy as jnp
from jax import lax
from jax.experimental import pallas as pl
from jax.experimental.pallas import tpu as pltpu
from jax.experimental.pallas import tpu_sc as plsc

N_FACTORS = 64
BATCH = 16384

_info = plsc.get_sparse_core_info()
_NC, _NS, _L = _info.num_cores, _info.num_subcores, _info.num_lanes
_NW = _NC * _NS
_BPW = BATCH // _NW


def _mf_body(user_hbm, movie_hbm, uf_hbm, mf_hbm, ub_hbm, mb_hbm, out_hbm,
             uidx_v, midx_v, urows_v, mrows_v, ub_v, mb_v, out_v, pacc_v,
             sem_u, sem_m, sem_ub, sem_mb):
    wid = lax.axis_index("s") * _NC + lax.axis_index("c")
    base = wid * _BPW

    pltpu.sync_copy(user_hbm.at[pl.ds(base, _BPW)], uidx_v)
    pltpu.sync_copy(movie_hbm.at[pl.ds(base, _BPW)], midx_v)

    cu = pltpu.make_async_copy(uf_hbm.at[uidx_v], urows_v, sem_u)
    cm = pltpu.make_async_copy(mf_hbm.at[midx_v], mrows_v, sem_m)
    cub = pltpu.make_async_copy(ub_hbm.at[uidx_v], ub_v, sem_ub)
    cmb = pltpu.make_async_copy(mb_hbm.at[midx_v], mb_v, sem_mb)
    cu.start(); cm.start(); cub.start(); cmb.start()
    cu.wait(); cm.wait(); cub.wait(); cmb.wait()

    lanes = lax.iota(jnp.int32, _L)

    def block(b, _):
        r0 = b * _L
        for r in range(_L):
            acc = urows_v[r0 + r, pl.ds(0, _L)] * mrows_v[r0 + r, pl.ds(0, _L)]
            for j in range(1, N_FACTORS // _L):
                acc = acc + (urows_v[r0 + r, pl.ds(j * _L, _L)]
                             * mrows_v[r0 + r, pl.ds(j * _L, _L)])
            pacc_v[pl.ds(r * _L, _L)] = acc
        tot = plsc.load_gather(pacc_v, [lanes * _L])
        for l in range(1, _L):
            tot = tot + plsc.load_gather(pacc_v, [lanes * _L + l])
        sl = pl.ds(r0, _L)
        out_v[sl] = tot + ub_v[sl] + mb_v[sl]
        return 0

    lax.fori_loop(0, _BPW // _L, block, 0)
    pltpu.sync_copy(out_v, out_hbm.at[pl.ds(base, _BPW)])


@jax.jit
def kernel(user, movie, user_factors, movie_factors, user_biases, movie_biases):
    mesh = plsc.VectorSubcoreMesh(core_axis_name="c", subcore_axis_name="s")
    run = pl.kernel(
        _mf_body,
        out_type=jax.ShapeDtypeStruct((BATCH,), jnp.float32),
        mesh=mesh,
        compiler_params=pltpu.CompilerParams(
            needs_layout_passes=False, use_tc_tiling_on_sc=False),
        scratch_types=[
            pltpu.VMEM((_BPW,), jnp.int32),
            pltpu.VMEM((_BPW,), jnp.int32),
            pltpu.VMEM((_BPW, N_FACTORS), jnp.float32),
            pltpu.VMEM((_BPW, N_FACTORS), jnp.float32),
            pltpu.VMEM((_BPW,), jnp.float32),
            pltpu.VMEM((_BPW,), jnp.float32),
            pltpu.VMEM((_BPW,), jnp.float32),
            pltpu.VMEM((_L * _L,), jnp.float32),
            pltpu.SemaphoreType.DMA,
            pltpu.SemaphoreType.DMA,
            pltpu.SemaphoreType.DMA,
            pltpu.SemaphoreType.DMA,
        ],
    )
    return run(user, movie, user_factors, movie_factors,
               user_biases.reshape(-1), movie_biases.reshape(-1))


# speedup vs baseline: 1.3721x; 1.3721x over previous
import functools

import jax
import jax.numpy as jnp
from jax import lax
from jax.experimental import pallas as pl
from jax.experimental.pallas import tpu as pltpu
from jax.experimental.pallas import tpu_sc as plsc

N_FACTORS = 64
BATCH = 16384

_info = plsc.get_sparse_core_info()
_NC, _NS, _L = _info.num_cores, _info.num_subcores, _info.num_lanes
_NW = _NC * _NS
_BPW = BATCH // _NW


def _mf_body(user_hbm, movie_hbm, uf_hbm, mf_hbm, out_hbm,
             uidx_v, midx_v, urows_v, mrows_v, out_v, pacc_v,
             sem_u, sem_m):
    wid = lax.axis_index("s") * _NC + lax.axis_index("c")
    base = wid * _BPW

    pltpu.sync_copy(user_hbm.at[pl.ds(base, _BPW)], uidx_v)
    pltpu.sync_copy(movie_hbm.at[pl.ds(base, _BPW)], midx_v)

    lanes = lax.iota(jnp.int32, _L)

    def block(b, _):
        r0 = b * _L
        uvec = uidx_v[pl.ds(r0, _L)]
        mvec = midx_v[pl.ds(r0, _L)]
        for r in range(_L):
            pltpu.make_async_copy(uf_hbm.at[uvec[r]], urows_v.at[r], sem_u).start()
            pltpu.make_async_copy(mf_hbm.at[mvec[r]], mrows_v.at[r], sem_m).start()
        for r in range(_L):
            pltpu.make_async_copy(uf_hbm.at[uvec[r]], urows_v.at[r], sem_u).wait()
            pltpu.make_async_copy(mf_hbm.at[mvec[r]], mrows_v.at[r], sem_m).wait()

        for r in range(_L):
            acc = urows_v[r, pl.ds(0, _L)] * mrows_v[r, pl.ds(0, _L)]
            for j in range(1, N_FACTORS // _L):
                acc = acc + (urows_v[r, pl.ds(j * _L, _L)]
                             * mrows_v[r, pl.ds(j * _L, _L)])
            pacc_v[pl.ds(r * _L, _L)] = acc
        tot = plsc.load_gather(pacc_v, [lanes * _L])
        for l in range(1, _L):
            tot = tot + plsc.load_gather(pacc_v, [lanes * _L + l])
        out_v[pl.ds(r0, _L)] = tot
        return 0

    lax.fori_loop(0, _BPW // _L, block, 0)
    pltpu.sync_copy(out_v, out_hbm.at[pl.ds(base, _BPW)])


@jax.jit
def kernel(user, movie, user_factors, movie_factors, user_biases, movie_biases):
    mesh = plsc.VectorSubcoreMesh(core_axis_name="c", subcore_axis_name="s")
    run = pl.kernel(
        _mf_body,
        out_type=jax.ShapeDtypeStruct((BATCH,), jnp.float32),
        mesh=mesh,
        compiler_params=pltpu.CompilerParams(
            needs_layout_passes=False, use_tc_tiling_on_sc=True),
        scratch_types=[
            pltpu.VMEM((_BPW,), jnp.int32),
            pltpu.VMEM((_BPW,), jnp.int32),
            pltpu.VMEM((_L, N_FACTORS), jnp.float32),
            pltpu.VMEM((_L, N_FACTORS), jnp.float32),
            pltpu.VMEM((_BPW,), jnp.float32),
            pltpu.VMEM((_L * _L,), jnp.float32),
            pltpu.SemaphoreType.DMA,
            pltpu.SemaphoreType.DMA,
        ],
    )
    dots = run(user, movie, user_factors, movie_factors)
    ub = jnp.take(user_biases, user, axis=0)
    mb = jnp.take(movie_biases, movie, axis=0)
    return dots + jnp.squeeze(ub + mb, axis=1)
